# Initial kernel scaffold; baseline (speedup 1.0000x reference)
import jax
import jax.numpy as jnp
from jax.experimental import pallas as pl


def kernel(t, co_embeddings, divided, no_embeddings, unrelated_embeddings, hidden_state, w_ih, w_hh, b_ih, b_hh, wq, bq, wk, bk, wv, bv):
    raise NotImplementedError("write your pallas kernel here")


# speedup vs baseline: 1.7900x; 1.7900x over previous
import jax
import jax.numpy as jnp
from jax.experimental import pallas as pl

_N = 5000
_D = 128
_A = 32
_R1 = 1000
_R2 = 200
_NEGBIAS = -1e30


def _mm_nt(a, b):
    return jax.lax.dot_general(
        a, b, (((1,), (1,)), ((), ())), preferred_element_type=jnp.float32)


def _prep_kernel(co_ref, no_ref, un_ref, h_ref, dv_ref,
                 wih_ref, whh_ref, bih_ref, bhh_ref,
                 wq_ref, bq_ref, wk_ref, bk_ref, wv_ref, bv_ref,
                 hnew0_ref, qn_ref, qu_ref, kn_ref, ku_ref, vh_ref,
                 m1max_ref):
    i = pl.program_id(0)
    co = co_ref[...]
    h = h_ref[...]
    gi = _mm_nt(co, wih_ref[...]) + bih_ref[...]
    gh = _mm_nt(h, whh_ref[...]) + bhh_ref[...]
    r = jax.nn.sigmoid(gi[:, :_D] + gh[:, :_D])
    z = jax.nn.sigmoid(gi[:, _D:2 * _D] + gh[:, _D:2 * _D])
    n = jnp.tanh(gi[:, 2 * _D:] + r * gh[:, 2 * _D:])
    h_m1 = (1.0 - z) * n + z * h

    m1 = dv_ref[:, 0:1] > 0
    hnew0_ref[...] = jnp.where(m1, h_m1, 0.0)
    blockmax = jnp.max(jnp.where(m1, h_m1, -jnp.inf), axis=0, keepdims=True)

    @pl.when(i == 0)
    def _():
        m1max_ref[...] = jnp.full_like(m1max_ref[...], -jnp.inf)

    m1max_ref[...] = jnp.maximum(m1max_ref[...], blockmax)

    no = no_ref[...]
    un = un_ref[...]
    qn_ref[...] = _mm_nt(no, wq_ref[...]) + bq_ref[...]
    qu_ref[...] = _mm_nt(un, wq_ref[...]) + bq_ref[...]
    kn_ref[...] = _mm_nt(no, wk_ref[...]) + bk_ref[...]
    ku_ref[...] = _mm_nt(un, wk_ref[...]) + bk_ref[...]
    vh_ref[...] = _mm_nt(co, wv_ref[...]) + bv_ref[...]


def _att_kernel(qn_ref, qu_ref, dv_ref, hnew0_ref,
                kn_ref, ku_ref, vh_ref, bn_ref, bu_ref,
                hnew_ref, m23max_ref):
    i = pl.program_id(0)
    inv = jnp.float32(1.0) / jnp.sqrt(jnp.float32(_A))
    kn = kn_ref[...]
    ku = ku_ref[...]
    vh = vh_ref[...]
    bn = bn_ref[...]
    bu = bu_ref[...]

    def attend(qb):
        g1 = _mm_nt(qb, kn) * inv + bn
        g2 = _mm_nt(qb, ku) * inv + bu
        m = jnp.maximum(jnp.max(g1, axis=1, keepdims=True),
                        jnp.max(g2, axis=1, keepdims=True))
        p = jnp.exp(g1 - m) + jnp.exp(g2 - m)
        l = jnp.sum(p, axis=1, keepdims=True)
        att = jnp.dot(p, vh, preferred_element_type=jnp.float32) / l
        return jnp.tanh(att)

    out_n = attend(qn_ref[...])
    out_u = attend(qu_ref[...])

    m2 = dv_ref[:, 1:2] > 0
    m3 = dv_ref[:, 2:3] > 0
    hn = jnp.where(m2, out_n, hnew0_ref[...])
    hn = jnp.where(m3, out_u, hn)
    hnew_ref[...] = hn

    bm = jnp.maximum(
        jnp.max(jnp.where(m2, out_n, -jnp.inf), axis=0, keepdims=True),
        jnp.max(jnp.where(m3, out_u, -jnp.inf), axis=0, keepdims=True))

    @pl.when(i == 0)
    def _():
        m23max_ref[...] = jnp.full_like(m23max_ref[...], -jnp.inf)

    m23max_ref[...] = jnp.maximum(m23max_ref[...], bm)


def kernel(t, co_embeddings, divided, no_embeddings, unrelated_embeddings,
           hidden_state, w_ih, w_hh, b_ih, b_hh, wq, bq, wk, bk, wv, bv):
    f32 = jnp.float32
    h = (hidden_state if hidden_state is not None
         else jnp.zeros((_N, _D), dtype=co_embeddings.dtype))
    tpos = jnp.asarray(t) > 0
    dv_eff = jnp.where(tpos, divided, divided * jnp.array([1, 0, 0], divided.dtype))

    nblk1 = _N // _R1
    row_spec1 = lambda w: pl.BlockSpec((_R1, w), lambda i: (i, 0))
    full = lambda a: pl.BlockSpec(a.shape, lambda i: tuple(0 for _ in a.shape))

    bih2 = b_ih.reshape(1, -1)
    bhh2 = b_hh.reshape(1, -1)
    bq2 = bq.reshape(1, -1)
    bk2 = bk.reshape(1, -1)
    bv2 = bv.reshape(1, -1)

    hnew0, q_n, q_u, k_n, k_u, v_h, m1max = pl.pallas_call(
        _prep_kernel,
        grid=(nblk1,),
        in_specs=[row_spec1(_D), row_spec1(_D), row_spec1(_D), row_spec1(_D),
                  row_spec1(3),
                  full(w_ih), full(w_hh), full(bih2), full(bhh2),
                  full(wq), full(bq2), full(wk), full(bk2),
                  full(wv), full(bv2)],
        out_specs=[row_spec1(_D), row_spec1(_A), row_spec1(_A),
                   row_spec1(_A), row_spec1(_A), row_spec1(_D),
                   pl.BlockSpec((1, _D), lambda i: (0, 0))],
        out_shape=[
            jax.ShapeDtypeStruct((_N, _D), f32),
            jax.ShapeDtypeStruct((_N, _A), f32),
            jax.ShapeDtypeStruct((_N, _A), f32),
            jax.ShapeDtypeStruct((_N, _A), f32),
            jax.ShapeDtypeStruct((_N, _A), f32),
            jax.ShapeDtypeStruct((_N, _D), f32),
            jax.ShapeDtypeStruct((1, _D), f32),
        ],
    )(co_embeddings, no_embeddings, unrelated_embeddings, h, dv_eff,
      w_ih, w_hh, bih2, bhh2, wq, bq2, wk, bk2, wv, bv2)

    bias_n = jnp.where(dv_eff[:, 1] > 0, f32(0), f32(_NEGBIAS)).reshape(1, _N)
    bias_u = jnp.where(dv_eff[:, 2] > 0, f32(0), f32(_NEGBIAS)).reshape(1, _N)

    nblk2 = _N // _R2
    row_spec2 = lambda w: pl.BlockSpec((_R2, w), lambda i: (i, 0))

    h_new, m23max = pl.pallas_call(
        _att_kernel,
        grid=(nblk2,),
        in_specs=[row_spec2(_A), row_spec2(_A), row_spec2(3), row_spec2(_D),
                  full(k_n), full(k_u), full(v_h), full(bias_n), full(bias_u)],
        out_specs=[row_spec2(_D), pl.BlockSpec((1, _D), lambda i: (0, 0))],
        out_shape=[
            jax.ShapeDtypeStruct((_N, _D), f32),
            jax.ShapeDtypeStruct((1, _D), f32),
        ],
    )(q_n, q_u, dv_eff, hnew0, k_n, k_u, v_h, bias_n, bias_u)

    out_m1 = m1max[0]
    out_m23 = m23max[0]
    has1 = jnp.isfinite(jnp.max(out_m1))
    has23 = jnp.isfinite(jnp.max(out_m23))
    output = jnp.where(~has1, out_m23,
                       jnp.where(~has23, out_m1,
                                 jnp.maximum(out_m1, out_m23)))
    return (output, h_new)


# speedup vs baseline: 2.4563x; 1.3722x over previous
import jax
import jax.numpy as jnp
from jax.experimental import pallas as pl

_N = 5000
_D = 128
_A = 32
_R1 = 1000
_R2 = 200
_NEGBIAS = -1e30


def _mm_nt(a, b):
    return jax.lax.dot_general(
        a, b, (((1,), (1,)), ((), ())), preferred_element_type=jnp.float32)


def _prep_kernel(co_ref, no_ref, un_ref, h_ref, dv_ref,
                 wih_ref, whh_ref, bih_ref, bhh_ref,
                 wq_ref, bq_ref, wk_ref, bk_ref, wv_ref, bv_ref,
                 hnew0_ref, qn_ref, qu_ref, kn_ref, ku_ref, vh_ref,
                 m1max_ref, kmax2_ref):
    i = pl.program_id(0)
    co = co_ref[...]
    h = h_ref[...]
    gi = _mm_nt(co, wih_ref[...]) + bih_ref[...]
    gh = _mm_nt(h, whh_ref[...]) + bhh_ref[...]
    r = jax.nn.sigmoid(gi[:, :_D] + gh[:, :_D])
    z = jax.nn.sigmoid(gi[:, _D:2 * _D] + gh[:, _D:2 * _D])
    n = jnp.tanh(gi[:, 2 * _D:] + r * gh[:, 2 * _D:])
    h_m1 = (1.0 - z) * n + z * h

    m1 = dv_ref[:, 0:1] > 0
    hnew0_ref[...] = jnp.where(m1, h_m1, 0.0)
    blockmax = jnp.max(jnp.where(m1, h_m1, -jnp.inf), axis=0, keepdims=True)

    @pl.when(i == 0)
    def _():
        m1max_ref[...] = jnp.full_like(m1max_ref[...], -jnp.inf)
        kmax2_ref[...] = jnp.zeros_like(kmax2_ref[...])

    m1max_ref[...] = jnp.maximum(m1max_ref[...], blockmax)

    no = no_ref[...]
    un = un_ref[...]
    bf16 = jnp.bfloat16
    kn = _mm_nt(no, wk_ref[...]) + bk_ref[...]
    ku = _mm_nt(un, wk_ref[...]) + bk_ref[...]
    qn_ref[...] = (_mm_nt(no, wq_ref[...]) + bq_ref[...]).astype(bf16)
    qu_ref[...] = (_mm_nt(un, wq_ref[...]) + bq_ref[...]).astype(bf16)
    kn16 = kn.astype(bf16)
    ku16 = ku.astype(bf16)
    kn_ref[...] = kn16
    ku_ref[...] = ku16
    vh_ref[...] = (_mm_nt(co, wv_ref[...]) + bv_ref[...]).astype(bf16)

    knf = kn16.astype(jnp.float32)
    kuf = ku16.astype(jnp.float32)
    k2 = jnp.maximum(jnp.max(jnp.sum(knf * knf, axis=1)),
                     jnp.max(jnp.sum(kuf * kuf, axis=1)))
    kmax2_ref[...] = jnp.maximum(kmax2_ref[...], k2)


def _att_kernel(qn_ref, qu_ref, dv_ref, hnew0_ref,
                kn_ref, ku_ref, vh_ref, bn_ref, bu_ref, kmax2_ref,
                hnew_ref, m23max_ref):
    i = pl.program_id(0)
    inv = jnp.float32(1.0) / jnp.sqrt(jnp.float32(_A))
    kn = kn_ref[...]
    ku = ku_ref[...]
    vh = vh_ref[...]
    bn = bn_ref[...]
    bu = bu_ref[...]
    km2 = kmax2_ref[0:1, 0:1]

    def attend(qb):
        qf = qb.astype(jnp.float32)
        q2 = jnp.sum(qf * qf, axis=1, keepdims=True)
        m = jnp.sqrt(q2 * km2) * inv + jnp.float32(1e-3)
        p = (jnp.exp(_mm_nt(qb, kn) * inv + (bn - m)) +
             jnp.exp(_mm_nt(qb, ku) * inv + (bu - m)))
        l = jnp.sum(p, axis=1, keepdims=True)
        att = jnp.dot(p.astype(jnp.bfloat16), vh,
                      preferred_element_type=jnp.float32) / l
        return jnp.tanh(att)

    out_n = attend(qn_ref[...])
    out_u = attend(qu_ref[...])

    m2 = dv_ref[:, 1:2] > 0
    m3 = dv_ref[:, 2:3] > 0
    hn = jnp.where(m2, out_n, hnew0_ref[...])
    hn = jnp.where(m3, out_u, hn)
    hnew_ref[...] = hn

    bm = jnp.maximum(
        jnp.max(jnp.where(m2, out_n, -jnp.inf), axis=0, keepdims=True),
        jnp.max(jnp.where(m3, out_u, -jnp.inf), axis=0, keepdims=True))

    @pl.when(i == 0)
    def _():
        m23max_ref[...] = jnp.full_like(m23max_ref[...], -jnp.inf)

    m23max_ref[...] = jnp.maximum(m23max_ref[...], bm)


def kernel(t, co_embeddings, divided, no_embeddings, unrelated_embeddings,
           hidden_state, w_ih, w_hh, b_ih, b_hh, wq, bq, wk, bk, wv, bv):
    f32 = jnp.float32
    bf16 = jnp.bfloat16
    h = (hidden_state if hidden_state is not None
         else jnp.zeros((_N, _D), dtype=co_embeddings.dtype))
    tpos = jnp.asarray(t) > 0
    dv_eff = jnp.where(tpos, divided, divided * jnp.array([1, 0, 0], divided.dtype))

    nblk1 = _N // _R1
    row_spec1 = lambda w: pl.BlockSpec((_R1, w), lambda i: (i, 0))
    full = lambda a: pl.BlockSpec(a.shape, lambda i: tuple(0 for _ in a.shape))

    bih2 = b_ih.reshape(1, -1)
    bhh2 = b_hh.reshape(1, -1)
    bq2 = bq.reshape(1, -1)
    bk2 = bk.reshape(1, -1)
    bv2 = bv.reshape(1, -1)

    hnew0, q_n, q_u, k_n, k_u, v_h, m1max, kmax2 = pl.pallas_call(
        _prep_kernel,
        grid=(nblk1,),
        in_specs=[row_spec1(_D), row_spec1(_D), row_spec1(_D), row_spec1(_D),
                  row_spec1(3),
                  full(w_ih), full(w_hh), full(bih2), full(bhh2),
                  full(wq), full(bq2), full(wk), full(bk2),
                  full(wv), full(bv2)],
        out_specs=[row_spec1(_D), row_spec1(_A), row_spec1(_A),
                   row_spec1(_A), row_spec1(_A), row_spec1(_D),
                   pl.BlockSpec((1, _D), lambda i: (0, 0)),
                   pl.BlockSpec((1, _D), lambda i: (0, 0))],
        out_shape=[
            jax.ShapeDtypeStruct((_N, _D), f32),
            jax.ShapeDtypeStruct((_N, _A), bf16),
            jax.ShapeDtypeStruct((_N, _A), bf16),
            jax.ShapeDtypeStruct((_N, _A), bf16),
            jax.ShapeDtypeStruct((_N, _A), bf16),
            jax.ShapeDtypeStruct((_N, _D), bf16),
            jax.ShapeDtypeStruct((1, _D), f32),
            jax.ShapeDtypeStruct((1, _D), f32),
        ],
    )(co_embeddings, no_embeddings, unrelated_embeddings, h, dv_eff,
      w_ih, w_hh, bih2, bhh2, wq, bq2, wk, bk2, wv, bv2)

    bias_n = jnp.where(dv_eff[:, 1] > 0, f32(0), f32(_NEGBIAS)).reshape(1, _N)
    bias_u = jnp.where(dv_eff[:, 2] > 0, f32(0), f32(_NEGBIAS)).reshape(1, _N)

    nblk2 = _N // _R2
    row_spec2 = lambda w: pl.BlockSpec((_R2, w), lambda i: (i, 0))

    h_new, m23max = pl.pallas_call(
        _att_kernel,
        grid=(nblk2,),
        in_specs=[row_spec2(_A), row_spec2(_A), row_spec2(3), row_spec2(_D),
                  full(k_n), full(k_u), full(v_h), full(bias_n), full(bias_u),
                  full(kmax2)],
        out_specs=[row_spec2(_D), pl.BlockSpec((1, _D), lambda i: (0, 0))],
        out_shape=[
            jax.ShapeDtypeStruct((_N, _D), f32),
            jax.ShapeDtypeStruct((1, _D), f32),
        ],
    )(q_n, q_u, dv_eff, hnew0, k_n, k_u, v_h, bias_n, bias_u, kmax2)

    out_m1 = m1max[0]
    out_m23 = m23max[0]
    has1 = jnp.isfinite(jnp.max(out_m1))
    has23 = jnp.isfinite(jnp.max(out_m23))
    output = jnp.where(~has1, out_m23,
                       jnp.where(~has23, out_m1,
                                 jnp.maximum(out_m1, out_m23)))
    return (output, h_new)
